# Initial kernel scaffold; baseline (speedup 1.0000x reference)
import jax
import jax.numpy as jnp
from jax.experimental import pallas as pl


def kernel(x, y, gamma, beta, w, wb):
    raise NotImplementedError("write your pallas kernel here")


# speedup vs baseline: 2.6145x; 2.6145x over previous
import jax
import jax.numpy as jnp
from jax.experimental import pallas as pl
from jax.experimental.pallas import tpu as pltpu

LN_EPS = 1e-5

TM = 128
NJ = 5
CORES = 2


def _fused_head_kernel(x_ref, y_ref, g_ref, b_ref, wb_ref, w_hbm,
                       out_ref, w_vmem, pred_ref, h_ref, m_ref, l_ref, sems):
    i1 = pl.program_id(1)
    jj = pl.program_id(2)
    nj = pl.num_programs(2) // 2
    tn = w_vmem.shape[-1]

    @pl.when((i1 == 0) & (jj == 0))
    def _start_w_copies():
        for c in range(NJ):
            pltpu.make_async_copy(
                w_hbm.at[:, c * tn:(c + 1) * tn], w_vmem.at[c], sems.at[c]
            ).start()

    @pl.when(jj == 0)
    def _layernorm():
        s = x_ref[...].astype(jnp.float32) + y_ref[...].astype(jnp.float32)
        mean = jnp.mean(s, axis=-1, keepdims=True)
        sc = s - mean
        var = jnp.mean(sc * sc, axis=-1, keepdims=True)
        sn = sc * jax.lax.rsqrt(var + LN_EPS)
        h_ref[...] = (sn * g_ref[...] + b_ref[...]).astype(h_ref.dtype)
        m_ref[...] = jnp.full(m_ref.shape, -jnp.inf, jnp.float32)
        l_ref[...] = jnp.zeros(l_ref.shape, jnp.float32)

    for c in range(NJ):
        @pl.when((i1 == 0) & (jj == c))
        def _wait_w(c=c):
            pltpu.make_async_copy(
                w_hbm.at[:, c * tn:(c + 1) * tn], w_vmem.at[c], sems.at[c]
            ).wait()

    @pl.when(jj < nj)
    def _phase1():
        c = jj
        p = jnp.dot(h_ref[...], w_vmem[c],
                    preferred_element_type=jnp.float32) + wb_ref[c]
        pred_ref[c] = p
        m_prev = m_ref[...]
        m_new = jnp.maximum(m_prev, jnp.max(p, axis=-1, keepdims=True))
        l_ref[...] = (l_ref[...] * jnp.exp(m_prev - m_new)
                      + jnp.sum(jnp.exp(p - m_new), axis=-1, keepdims=True))
        m_ref[...] = m_new

    @pl.when(jj >= nj)
    def _phase2():
        lse = m_ref[...] + jnp.log(l_ref[...])
        out_ref[...] = pred_ref[jj - nj] - lse


def kernel(x, y, gamma, beta, w, wb):
    R, D = x.shape
    V = w.shape[1]
    tn = V // NJ
    blocks_per_core = R // TM // CORES

    w_bf = w.astype(jnp.bfloat16)
    wb3 = wb.reshape(NJ, 1, tn).astype(jnp.float32)

    nbytes = (NJ * D * tn * 2
              + NJ * TM * tn * 4
              + 2 * TM * tn * 4
              + 4 * TM * D * 4
              + 2 * NJ * tn * 4
              + TM * D * 2 + 6 * D * 4 + 2 * TM * 4 + (1 << 20))

    out = pl.pallas_call(
        _fused_head_kernel,
        out_shape=jax.ShapeDtypeStruct((R, V), jnp.float32),
        grid_spec=pltpu.PrefetchScalarGridSpec(
            num_scalar_prefetch=0,
            grid=(CORES, blocks_per_core, 2 * NJ),
            in_specs=[
                pl.BlockSpec((TM, D),
                             lambda i0, i1, jj: (i0 * blocks_per_core + i1, 0)),
                pl.BlockSpec((TM, D),
                             lambda i0, i1, jj: (i0 * blocks_per_core + i1, 0)),
                pl.BlockSpec((1, D), lambda i0, i1, jj: (0, 0)),
                pl.BlockSpec((1, D), lambda i0, i1, jj: (0, 0)),
                pl.BlockSpec((NJ, 1, tn), lambda i0, i1, jj: (0, 0, 0)),
                pl.BlockSpec(memory_space=pl.ANY),
            ],
            out_specs=pl.BlockSpec(
                (TM, tn),
                lambda i0, i1, jj: (i0 * blocks_per_core + i1,
                                    jnp.maximum(jj - NJ, 0))),
            scratch_shapes=[
                pltpu.VMEM((NJ, D, tn), jnp.bfloat16),
                pltpu.VMEM((NJ, TM, tn), jnp.float32),
                pltpu.VMEM((TM, D), jnp.bfloat16),
                pltpu.VMEM((TM, 1), jnp.float32),
                pltpu.VMEM((TM, 1), jnp.float32),
                pltpu.SemaphoreType.DMA((NJ,)),
            ],
        ),
        compiler_params=pltpu.CompilerParams(
            dimension_semantics=("parallel", "arbitrary", "arbitrary"),
            vmem_limit_bytes=min(nbytes, 56 * 1024 * 1024),
        ),
    )(x, y, gamma, beta, wb3, w_bf)
    return out


# speedup vs baseline: 2.9107x; 1.1133x over previous
import jax
import jax.numpy as jnp
from jax.experimental import pallas as pl
from jax.experimental.pallas import tpu as pltpu

LN_EPS = 1e-5

TM = 128
ND = 5
NC = 25


def _fused_head_kernel(x_ref, y_ref, g_ref, b_ref, wb_ref, w_hbm,
                       out_ref, w_vmem, pred_ref, lse_ref, sems):
    i = pl.program_id(0)
    jj = pl.program_id(1)
    V = pred_ref.shape[-1]
    td = w_vmem.shape[-1]
    tc = V // NC
    sub_per_dma = NC // ND

    @pl.when((i == 0) & (jj == 0))
    def _start_w_copies():
        for d in range(ND):
            pltpu.make_async_copy(
                w_hbm.at[:, d * td:(d + 1) * td], w_vmem.at[d], sems.at[d]
            ).start()

    @pl.when(jj == 0)
    def _phase1():
        s = x_ref[...].astype(jnp.float32) + y_ref[...].astype(jnp.float32)
        mean = jnp.mean(s, axis=-1, keepdims=True)
        sc = s - mean
        var = jnp.mean(sc * sc, axis=-1, keepdims=True)
        sn = sc * jax.lax.rsqrt(var + LN_EPS)
        h = (sn * g_ref[...] + b_ref[...]).astype(jnp.bfloat16)

        m = jnp.full((TM, 1), -jnp.inf, jnp.float32)
        l = jnp.zeros((TM, 1), jnp.float32)
        for c in range(NC):
            d, r = divmod(c, sub_per_dma)
            if r == 0:
                @pl.when(i == 0)
                def _wait_w(d=d):
                    pltpu.make_async_copy(
                        w_hbm.at[:, d * td:(d + 1) * td], w_vmem.at[d],
                        sems.at[d]).wait()
            p = jnp.dot(h, w_vmem[d][:, r * tc:(r + 1) * tc],
                        preferred_element_type=jnp.float32) + wb_ref[c]
            pred_ref[:, c * tc:(c + 1) * tc] = p
            m_new = jnp.maximum(m, jnp.max(p, axis=-1, keepdims=True))
            l = l * jnp.exp(m - m_new) + jnp.sum(jnp.exp(p - m_new),
                                                 axis=-1, keepdims=True)
            m = m_new
        lse_ref[...] = m + jnp.log(l)

    tn = out_ref.shape[-1]
    for c2 in range(V // tn):
        @pl.when(jj == c2 + 1)
        def _phase2(c2=c2):
            out_ref[...] = pred_ref[:, c2 * tn:(c2 + 1) * tn] - lse_ref[...]


def kernel(x, y, gamma, beta, w, wb):
    R, D = x.shape
    V = w.shape[1]
    td = V // ND
    tn = V // ND

    w_bf = w.astype(jnp.bfloat16)
    wb3 = wb.reshape(NC, 1, V // NC).astype(jnp.float32)

    out = pl.pallas_call(
        _fused_head_kernel,
        out_shape=jax.ShapeDtypeStruct((R, V), jnp.float32),
        grid_spec=pltpu.PrefetchScalarGridSpec(
            num_scalar_prefetch=0,
            grid=(R // TM, 1 + V // tn),
            in_specs=[
                pl.BlockSpec((TM, D), lambda i, jj: (i, 0)),
                pl.BlockSpec((TM, D), lambda i, jj: (i, 0)),
                pl.BlockSpec((1, D), lambda i, jj: (0, 0)),
                pl.BlockSpec((1, D), lambda i, jj: (0, 0)),
                pl.BlockSpec((NC, 1, V // NC), lambda i, jj: (0, 0, 0)),
                pl.BlockSpec(memory_space=pl.ANY),
            ],
            out_specs=pl.BlockSpec(
                (TM, tn), lambda i, jj: (i, jnp.maximum(jj - 1, 0))),
            scratch_shapes=[
                pltpu.VMEM((ND, D, td), jnp.bfloat16),
                pltpu.VMEM((TM, V), jnp.float32),
                pltpu.VMEM((TM, 1), jnp.float32),
                pltpu.SemaphoreType.DMA((ND,)),
            ],
        ),
        compiler_params=pltpu.CompilerParams(
            dimension_semantics=("parallel", "arbitrary"),
            vmem_limit_bytes=59904 * 1024,
        ),
    )(x, y, gamma, beta, wb3, w_bf)
    return out
